# baseline (speedup 1.0000x reference)
import jax
import jax.numpy as jnp
from jax import lax
from jax.experimental import pallas as pl
from jax.experimental.pallas import tpu as pltpu
from jax.experimental.pallas import tpu_sc as plsc

_B = 16384
_V = 33
_DIN = 16
_DOUT = 128

_C = 4096
_G = _B // _C


def _tc_body(inpT_ref, L0_ref, fs_ref, b_ref, vuT_ref,
             out_ref, hgT_ref, lhT_ref):
    Lrow = L0_ref[...]
    m = jnp.max(Lrow, axis=1, keepdims=True)
    e = jnp.exp(Lrow - m)
    w = e / jnp.sum(e, axis=1, keepdims=True)
    fc = lax.dot_general(w, fs_ref[...], (((1,), (0,)), ((), ())),
                         preferred_element_type=jnp.float32) + b_ref[...]
    hcT = lax.dot_general(vuT_ref[...], w, (((1,), (1,)), ((), ())),
                          preferred_element_type=jnp.float32)
    wT = w.reshape(_V, 1)
    rwT = 1.0 / (wT + 1e-20)

    xT = inpT_ref[...]
    s = jnp.sum(xT, axis=0, keepdims=True)
    iwT = jnp.concatenate([1.0 - s, xT], axis=0)
    wdT = iwT * rwT
    val = jnp.min(wdT, axis=0, keepdims=True)
    MT = jnp.where(wdT == val, 0.0, iwT - val * wT)
    MT_full = jnp.concatenate([MT, val], axis=0)
    fs_full = jnp.concatenate([fs_ref[...], fc], axis=0)
    out = lax.dot_general(MT_full, fs_full, (((0,), (0,)), ((), ())),
                          preferred_element_type=jnp.float32)
    out_ref[...] = out
    hgT_ref[...] = jnp.broadcast_to(hcT, (_DIN, _C))
    lhT_ref[...] = jnp.broadcast_to(hcT, (_DIN, _C))


_NC = 2
_NS = 16
_NW = _NC * _NS
_COLS_W = _B // _NW
_QC = 256


def _sc_body(hpat_hbm, holdT_hbm, shared, sem):
    cid = lax.axis_index("c")
    sid = lax.axis_index("s")
    wid = sid * _NC + cid
    base = wid * _COLS_W

    @pl.when(sid == 0)
    def _load():
        pltpu.sync_copy(hpat_hbm, shared)

    plsc.subcore_barrier()
    handles = [
        pltpu.async_copy(shared, holdT_hbm.at[:, pl.ds(base + j * _QC, _QC)],
                         sem)
        for j in range(_COLS_W // _QC)
    ]
    for h in handles:
        h.wait()


def _sc_holdT(visible_units):
    mesh = plsc.VectorSubcoreMesh(core_axis_name="c", subcore_axis_name="s")
    kern = pl.kernel(
        _sc_body,
        out_type=jax.ShapeDtypeStruct((_V * _DIN, _B), jnp.float32),
        mesh=mesh,
        scratch_types=[
            pltpu.VMEM_SHARED((_V * _DIN, _QC), jnp.float32),
            pltpu.SemaphoreType.DMA,
        ],
    )
    hpat = jnp.broadcast_to(visible_units.reshape(_V * _DIN, 1), (_V * _DIN, _QC))
    return kern(hpat)


def kernel(inp, L0, visible_fs, biases, visible_units):
    out, hgT, lhT = pl.pallas_call(
        _tc_body,
        grid=(_G,),
        in_specs=[
            pl.BlockSpec((_V - 1, _C), lambda i: (0, i)),
            pl.BlockSpec((1, _V), lambda i: (0, 0)),
            pl.BlockSpec((_V, _DOUT), lambda i: (0, 0)),
            pl.BlockSpec((1, _DOUT), lambda i: (0, 0)),
            pl.BlockSpec((_DIN, _V), lambda i: (0, 0)),
        ],
        out_specs=[
            pl.BlockSpec((_C, _DOUT), lambda i: (i, 0)),
            pl.BlockSpec((_DIN, _C), lambda i: (0, i)),
            pl.BlockSpec((_DIN, _C), lambda i: (0, i)),
        ],
        out_shape=[
            jax.ShapeDtypeStruct((_B, _DOUT), jnp.float32),
            jax.ShapeDtypeStruct((_DIN, _B), jnp.float32),
            jax.ShapeDtypeStruct((_DIN, _B), jnp.float32),
        ],
    )(inp.T, L0, visible_fs, biases, visible_units.T)

    holdT = _sc_holdT(visible_units)

    hold = holdT.reshape(_V, _DIN, _B).transpose(2, 0, 1)
    return (out.reshape(_B, 1, _DOUT), hold, hgT.T, lhT.T)


# speedup vs baseline: 1.0288x; 1.0288x over previous
import jax
import jax.numpy as jnp
from jax import lax
from jax.experimental import pallas as pl
from jax.experimental.pallas import tpu as pltpu
from jax.experimental.pallas import tpu_sc as plsc

_B = 16384
_V = 33
_DIN = 16
_DOUT = 128

_C = 1024
_G = _B // _C


def _tc_body(inpT_ref, L0_ref, fs_ref, b_ref, vuT_ref,
             out_ref, hgT_ref, lhT_ref):
    Lrow = L0_ref[...]
    m = jnp.max(Lrow, axis=1, keepdims=True)
    e = jnp.exp(Lrow - m)
    w = e / jnp.sum(e, axis=1, keepdims=True)
    fc = lax.dot_general(w, fs_ref[...], (((1,), (0,)), ((), ())),
                         preferred_element_type=jnp.float32) + b_ref[...]
    hcT = lax.dot_general(vuT_ref[...], w, (((1,), (1,)), ((), ())),
                          preferred_element_type=jnp.float32)
    wT = w.reshape(_V, 1)
    rwT = 1.0 / (wT + 1e-20)

    xT = inpT_ref[...]
    s = jnp.sum(xT, axis=0, keepdims=True)
    iwT = jnp.concatenate([1.0 - s, xT], axis=0)
    wdT = iwT * rwT
    val = jnp.min(wdT, axis=0, keepdims=True)
    MT = jnp.where(wdT == val, 0.0, iwT - val * wT)
    MT_full = jnp.concatenate([MT, val], axis=0)
    fs_full = jnp.concatenate([fs_ref[...], fc], axis=0)
    out = lax.dot_general(MT_full, fs_full, (((0,), (0,)), ((), ())),
                          preferred_element_type=jnp.float32)
    out_ref[...] = out
    hgT_ref[...] = jnp.broadcast_to(hcT, (_DIN, _C))
    lhT_ref[...] = jnp.broadcast_to(hcT, (_DIN, _C))


_NC = 2
_NS = 16
_NW = _NC * _NS
_COLS_W = _B // _NW
_QC = 128


def _sc_body(hpat_hbm, holdT_hbm, shared, sem):
    cid = lax.axis_index("c")
    sid = lax.axis_index("s")
    wid = sid * _NC + cid
    base = wid * _COLS_W

    @pl.when(sid == 0)
    def _load():
        pltpu.sync_copy(hpat_hbm, shared)

    plsc.subcore_barrier()
    handles = [
        pltpu.async_copy(shared, holdT_hbm.at[:, pl.ds(base + j * _QC, _QC)],
                         sem)
        for j in range(_COLS_W // _QC)
    ]
    for h in handles:
        h.wait()


def _sc_holdT(visible_units):
    mesh = plsc.VectorSubcoreMesh(core_axis_name="c", subcore_axis_name="s")
    kern = pl.kernel(
        _sc_body,
        out_type=jax.ShapeDtypeStruct((_V * _DIN, _B), jnp.float32),
        mesh=mesh,
        scratch_types=[
            pltpu.VMEM_SHARED((_V * _DIN, _QC), jnp.float32),
            pltpu.SemaphoreType.DMA,
        ],
    )
    hpat = jnp.broadcast_to(visible_units.reshape(_V * _DIN, 1), (_V * _DIN, _QC))
    return kern(hpat)


def kernel(inp, L0, visible_fs, biases, visible_units):
    out, hgT, lhT = pl.pallas_call(
        _tc_body,
        grid=(_G,),
        in_specs=[
            pl.BlockSpec((_V - 1, _C), lambda i: (0, i)),
            pl.BlockSpec((1, _V), lambda i: (0, 0)),
            pl.BlockSpec((_V, _DOUT), lambda i: (0, 0)),
            pl.BlockSpec((1, _DOUT), lambda i: (0, 0)),
            pl.BlockSpec((_DIN, _V), lambda i: (0, 0)),
        ],
        out_specs=[
            pl.BlockSpec((_C, _DOUT), lambda i: (i, 0)),
            pl.BlockSpec((_DIN, _C), lambda i: (0, i)),
            pl.BlockSpec((_DIN, _C), lambda i: (0, i)),
        ],
        out_shape=[
            jax.ShapeDtypeStruct((_B, _DOUT), jnp.float32),
            jax.ShapeDtypeStruct((_DIN, _B), jnp.float32),
            jax.ShapeDtypeStruct((_DIN, _B), jnp.float32),
        ],
    )(inp.T, L0, visible_fs, biases, visible_units.T)

    holdT = _sc_holdT(visible_units)

    hold = holdT.reshape(_V, _DIN, _B).transpose(2, 0, 1)
    return (out.reshape(_B, 1, _DOUT), hold, hgT.T, lhT.T)


# speedup vs baseline: 1.0303x; 1.0014x over previous
import jax
import jax.numpy as jnp
from jax import lax
from jax.experimental import pallas as pl
from jax.experimental.pallas import tpu as pltpu
from jax.experimental.pallas import tpu_sc as plsc

_B = 16384
_V = 33
_DIN = 16
_DOUT = 128

_C = 2048
_G = _B // _C


def _tc_body(inpT_ref, L0_ref, fs_ref, b_ref, vuT_ref,
             out_ref, hgT_ref, lhT_ref):
    Lrow = L0_ref[...]
    m = jnp.max(Lrow, axis=1, keepdims=True)
    e = jnp.exp(Lrow - m)
    w = e / jnp.sum(e, axis=1, keepdims=True)
    fc = lax.dot_general(w, fs_ref[...], (((1,), (0,)), ((), ())),
                         preferred_element_type=jnp.float32) + b_ref[...]
    hcT = lax.dot_general(vuT_ref[...], w, (((1,), (1,)), ((), ())),
                          preferred_element_type=jnp.float32)
    wT = w.reshape(_V, 1)
    rwT = 1.0 / (wT + 1e-20)

    xT = inpT_ref[...]
    s = jnp.sum(xT, axis=0, keepdims=True)
    iwT = jnp.concatenate([1.0 - s, xT], axis=0)
    wdT = iwT * rwT
    val = jnp.min(wdT, axis=0, keepdims=True)
    MT = jnp.where(wdT == val, 0.0, iwT - val * wT)
    MT_full = jnp.concatenate([MT, val], axis=0)
    fs_full = jnp.concatenate([fs_ref[...], fc], axis=0)
    out = lax.dot_general(MT_full, fs_full, (((0,), (0,)), ((), ())),
                          preferred_element_type=jnp.float32)
    out_ref[...] = out
    hgT_ref[...] = jnp.broadcast_to(hcT, (_DIN, _C))
    lhT_ref[...] = jnp.broadcast_to(hcT, (_DIN, _C))


_NC = 2
_NS = 16
_NW = _NC * _NS
_COLS_W = _B // _NW
_QC = 128


def _sc_body(hpat_hbm, holdT_hbm, shared, sem):
    cid = lax.axis_index("c")
    sid = lax.axis_index("s")
    wid = sid * _NC + cid
    base = wid * _COLS_W

    @pl.when(sid == 0)
    def _load():
        pltpu.sync_copy(hpat_hbm, shared)

    plsc.subcore_barrier()
    handles = [
        pltpu.async_copy(shared, holdT_hbm.at[:, pl.ds(base + j * _QC, _QC)],
                         sem)
        for j in range(_COLS_W // _QC)
    ]
    for h in handles:
        h.wait()


def _sc_holdT(visible_units):
    mesh = plsc.VectorSubcoreMesh(core_axis_name="c", subcore_axis_name="s")
    kern = pl.kernel(
        _sc_body,
        out_type=jax.ShapeDtypeStruct((_V * _DIN, _B), jnp.float32),
        mesh=mesh,
        scratch_types=[
            pltpu.VMEM_SHARED((_V * _DIN, _QC), jnp.float32),
            pltpu.SemaphoreType.DMA,
        ],
    )
    hpat = jnp.broadcast_to(visible_units.reshape(_V * _DIN, 1), (_V * _DIN, _QC))
    return kern(hpat)


def kernel(inp, L0, visible_fs, biases, visible_units):
    out, hgT, lhT = pl.pallas_call(
        _tc_body,
        grid=(_G,),
        in_specs=[
            pl.BlockSpec((_V - 1, _C), lambda i: (0, i)),
            pl.BlockSpec((1, _V), lambda i: (0, 0)),
            pl.BlockSpec((_V, _DOUT), lambda i: (0, 0)),
            pl.BlockSpec((1, _DOUT), lambda i: (0, 0)),
            pl.BlockSpec((_DIN, _V), lambda i: (0, 0)),
        ],
        out_specs=[
            pl.BlockSpec((_C, _DOUT), lambda i: (i, 0)),
            pl.BlockSpec((_DIN, _C), lambda i: (0, i)),
            pl.BlockSpec((_DIN, _C), lambda i: (0, i)),
        ],
        out_shape=[
            jax.ShapeDtypeStruct((_B, _DOUT), jnp.float32),
            jax.ShapeDtypeStruct((_DIN, _B), jnp.float32),
            jax.ShapeDtypeStruct((_DIN, _B), jnp.float32),
        ],
    )(inp.T, L0, visible_fs, biases, visible_units.T)

    holdT = _sc_holdT(visible_units)

    hold = holdT.reshape(_V, _DIN, _B).transpose(2, 0, 1)
    return (out.reshape(_B, 1, _DOUT), hold, hgT.T, lhT.T)
